# Initial kernel scaffold; baseline (speedup 1.0000x reference)
import functools

import jax
import jax.numpy as jnp
from jax import lax
from jax.experimental import pallas as pl
from jax.experimental.pallas import tpu as pltpu
from jax.experimental.pallas import tpu_sc as plsc

_NUM_WORKERS = 32
_ROWS_PER_DMA = 128


def kernel(x, table):
    batch, fields = x.shape
    depth = table.shape[1]
    total = batch * fields
    assert total % (_NUM_WORKERS * _ROWS_PER_DMA) == 0
    dmas_per_w = total // (_NUM_WORKERS * _ROWS_PER_DMA)

    idx2d = x.reshape(total // _ROWS_PER_DMA, _ROWS_PER_DMA).astype(jnp.int32)

    mesh = plsc.VectorSubcoreMesh(core_axis_name="c", subcore_axis_name="s")

    @functools.partial(
        pl.kernel,
        mesh=mesh,
        out_type=jax.ShapeDtypeStruct((total, depth), jnp.float32),
        scratch_types=[
            pltpu.VMEM((dmas_per_w, _ROWS_PER_DMA), jnp.int32),
            pltpu.VMEM((_ROWS_PER_DMA, depth), jnp.float32),
            pltpu.SemaphoreType.DMA,
        ],
    )
    def body(idx_hbm, table_hbm, out_hbm, idx_v, rows_v, sem):
        wid = lax.axis_index("s") * 2 + lax.axis_index("c")
        row0 = wid * dmas_per_w
        pltpu.sync_copy(idx_hbm.at[pl.ds(row0, dmas_per_w)], idx_v)

        def step(j, carry):
            pltpu.async_copy(table_hbm.at[idx_v.at[j]], rows_v, sem).wait()
            pltpu.sync_copy(
                rows_v,
                out_hbm.at[pl.ds((row0 + j) * _ROWS_PER_DMA, _ROWS_PER_DMA)],
            )
            return carry

        lax.fori_loop(0, dmas_per_w, step, 0)

    out = body(idx2d, table)
    return out.reshape(batch, fields, depth)


# speedup vs baseline: 1.4363x; 1.4363x over previous
import functools

import jax
import jax.numpy as jnp
from jax import lax
from jax.experimental import pallas as pl
from jax.experimental.pallas import tpu as pltpu
from jax.experimental.pallas import tpu_sc as plsc

_NUM_WORKERS = 32
_ROWS_PER_DMA = 128


def kernel(x, table):
    batch, fields = x.shape
    depth = table.shape[1]
    total = batch * fields
    assert total % (_NUM_WORKERS * _ROWS_PER_DMA) == 0
    dmas_per_w = total // (_NUM_WORKERS * _ROWS_PER_DMA)

    idx2d = x.reshape(total // _ROWS_PER_DMA, _ROWS_PER_DMA).astype(jnp.int32)

    mesh = plsc.VectorSubcoreMesh(core_axis_name="c", subcore_axis_name="s")

    @functools.partial(
        pl.kernel,
        mesh=mesh,
        compiler_params=pltpu.CompilerParams(use_tc_tiling_on_sc=False),
        out_type=jax.ShapeDtypeStruct((total, depth), jnp.float32),
        scratch_types=[
            pltpu.VMEM((dmas_per_w, _ROWS_PER_DMA), jnp.int32),
            pltpu.VMEM((_ROWS_PER_DMA, depth), jnp.float32),
            pltpu.SemaphoreType.DMA,
        ],
    )
    def body(idx_hbm, table_hbm, out_hbm, idx_v, rows_v, sem):
        wid = lax.axis_index("s") * 2 + lax.axis_index("c")
        row0 = wid * dmas_per_w
        pltpu.sync_copy(idx_hbm.at[pl.ds(row0, dmas_per_w)], idx_v)

        def step(j, carry):
            pltpu.async_copy(table_hbm.at[idx_v.at[j]], rows_v, sem).wait()
            pltpu.sync_copy(
                rows_v,
                out_hbm.at[pl.ds((row0 + j) * _ROWS_PER_DMA, _ROWS_PER_DMA)],
            )
            return carry

        lax.fori_loop(0, dmas_per_w, step, 0)

    out = body(idx2d, table)
    return out.reshape(batch, fields, depth)


# speedup vs baseline: 1.5664x; 1.0906x over previous
import functools

import jax
import jax.numpy as jnp
from jax import lax
from jax.experimental import pallas as pl
from jax.experimental.pallas import tpu as pltpu
from jax.experimental.pallas import tpu_sc as plsc

_NUM_WORKERS = 32
_ROWS_PER_DMA = 128
_K = 8


def kernel(x, table):
    batch, fields = x.shape
    depth = table.shape[1]
    total = batch * fields
    assert total % (_NUM_WORKERS * _ROWS_PER_DMA * _K) == 0
    dmas_per_w = total // (_NUM_WORKERS * _ROWS_PER_DMA)
    num_groups = dmas_per_w // _K
    group_rows = _K * _ROWS_PER_DMA

    idx2d = x.reshape(total // _ROWS_PER_DMA, _ROWS_PER_DMA).astype(jnp.int32)

    mesh = plsc.VectorSubcoreMesh(core_axis_name="c", subcore_axis_name="s")

    @functools.partial(
        pl.kernel,
        mesh=mesh,
        compiler_params=pltpu.CompilerParams(use_tc_tiling_on_sc=False),
        out_type=jax.ShapeDtypeStruct((total, depth), jnp.float32),
        scratch_types=[
            pltpu.VMEM((dmas_per_w, _ROWS_PER_DMA), jnp.int32),
            pltpu.VMEM((2 * group_rows, depth), jnp.float32),
            pltpu.SemaphoreType.DMA,
            pltpu.SemaphoreType.DMA,
            pltpu.SemaphoreType.DMA,
        ],
    )
    def body(idx_hbm, table_hbm, out_hbm, idx_v, buf, sem_g, sem_w0, sem_w1):
        wid = lax.axis_index("s") * 2 + lax.axis_index("c")
        row0 = wid * dmas_per_w
        pltpu.sync_copy(idx_hbm.at[pl.ds(row0, dmas_per_w)], idx_v)

        def write_desc(t, sem):
            slot = (t % 2) * group_rows
            return pltpu.make_async_copy(
                buf.at[pl.ds(slot, group_rows)],
                out_hbm.at[pl.ds((row0 + t * _K) * _ROWS_PER_DMA, group_rows)],
                sem,
            )

        def step(t, carry):
            slot = (t % 2) * group_rows
            parity = t % 2

            @pl.when(t >= 2)
            def _():
                @pl.when(parity == 0)
                def _():
                    write_desc(t - 2, sem_w0).wait()

                @pl.when(parity == 1)
                def _():
                    write_desc(t - 2, sem_w1).wait()

            descs = []
            for b in range(_K):
                descs.append(
                    pltpu.async_copy(
                        table_hbm.at[idx_v.at[t * _K + b]],
                        buf.at[pl.ds(slot + b * _ROWS_PER_DMA, _ROWS_PER_DMA)],
                        sem_g,
                    )
                )
            for d in descs:
                d.wait()

            @pl.when(parity == 0)
            def _():
                write_desc(t, sem_w0).start()

            @pl.when(parity == 1)
            def _():
                write_desc(t, sem_w1).start()

            return carry

        lax.fori_loop(0, num_groups, step, 0)
        if num_groups % 2 == 0:
            write_desc(num_groups - 2, sem_w0).wait()
            write_desc(num_groups - 1, sem_w1).wait()
        else:
            write_desc(num_groups - 2, sem_w1).wait()
            write_desc(num_groups - 1, sem_w0).wait()

    out = body(idx2d, table)
    return out.reshape(batch, fields, depth)


# speedup vs baseline: 1.5766x; 1.0065x over previous
import functools

import jax
import jax.numpy as jnp
from jax import lax
from jax.experimental import pallas as pl
from jax.experimental.pallas import tpu as pltpu
from jax.experimental.pallas import tpu_sc as plsc

_NUM_WORKERS = 32
_ROWS_PER_DMA = 128
_K = 8


def kernel(x, table):
    batch, fields = x.shape
    depth = table.shape[1]
    total = batch * fields
    assert total % (_NUM_WORKERS * _ROWS_PER_DMA * _K) == 0
    dmas_per_w = total // (_NUM_WORKERS * _ROWS_PER_DMA)
    num_groups = dmas_per_w // _K
    group_rows = _K * _ROWS_PER_DMA

    idx2d = x.reshape(total // _ROWS_PER_DMA, _ROWS_PER_DMA).astype(jnp.int32)

    mesh = plsc.VectorSubcoreMesh(core_axis_name="c", subcore_axis_name="s")

    @functools.partial(
        pl.kernel,
        mesh=mesh,
        compiler_params=pltpu.CompilerParams(use_tc_tiling_on_sc=False),
        out_type=jax.ShapeDtypeStruct((total, depth), jnp.float32),
        scratch_types=[
            pltpu.VMEM((dmas_per_w, _ROWS_PER_DMA), jnp.int32),
            pltpu.VMEM((2 * group_rows, depth), jnp.float32),
            pltpu.SemaphoreType.DMA,
            pltpu.SemaphoreType.DMA,
            pltpu.SemaphoreType.DMA,
            pltpu.SemaphoreType.DMA,
        ],
    )
    def body(idx_hbm, table_hbm, out_hbm, idx_v, buf,
             sem_g0, sem_g1, sem_w0, sem_w1):
        wid = lax.axis_index("s") * 2 + lax.axis_index("c")
        row0 = wid * dmas_per_w
        pltpu.sync_copy(idx_hbm.at[pl.ds(row0, dmas_per_w)], idx_v)

        def out_slice(t):
            return out_hbm.at[
                pl.ds((row0 + t * _K) * _ROWS_PER_DMA, group_rows)
            ]

        def buf_slot(t):
            return buf.at[pl.ds((t % 2) * group_rows, group_rows)]

        def fire_gathers(t, sem):
            slot = (t % 2) * group_rows
            for b in range(_K):
                pltpu.async_copy(
                    table_hbm.at[idx_v.at[t * _K + b]],
                    buf.at[pl.ds(slot + b * _ROWS_PER_DMA, _ROWS_PER_DMA)],
                    sem,
                )

        def drain_gathers(t, sem):
            pltpu.make_async_copy(out_slice(t), buf_slot(t), sem).wait()

        def write_group(t, sem):
            return pltpu.make_async_copy(buf_slot(t), out_slice(t), sem)

        fire_gathers(0, sem_g0)

        def step(t, carry):
            parity = t % 2

            @pl.when(parity == 0)
            def _():
                @pl.when(t >= 1)
                def _():
                    write_group(t - 1, sem_w1).wait()

                @pl.when(t + 1 < num_groups)
                def _():
                    fire_gathers(t + 1, sem_g1)

                drain_gathers(t, sem_g0)
                write_group(t, sem_w0).start()

            @pl.when(parity == 1)
            def _():
                write_group(t - 1, sem_w0).wait()

                @pl.when(t + 1 < num_groups)
                def _():
                    fire_gathers(t + 1, sem_g0)

                drain_gathers(t, sem_g1)
                write_group(t, sem_w1).start()

            return carry

        lax.fori_loop(0, num_groups, step, 0)
        last_sem = sem_w0 if (num_groups - 1) % 2 == 0 else sem_w1
        write_group(num_groups - 1, last_sem).wait()

    out = body(idx2d, table)
    return out.reshape(batch, fields, depth)
